# baseline (speedup 1.0000x reference)
import jax
import jax.numpy as jnp
from jax import lax
from jax.experimental import pallas as pl
from jax.experimental.pallas import tpu as pltpu
from jax.experimental.pallas import tpu_sc as plsc

GRID = 24
RES = 0.25
HALF = GRID * RES / 2.0
PG = GRID + 2
PC = PG * PG
ROWS = 680
ACCN = ROWS * 128
NPTS = 16 * 2048
NWORK = 32
PPW = NPTS // NWORK

SHIFTS = [(kh - 1) * PG + (kw - 1) for kh in range(3) for kw in range(3)]


SMCH = 512


def _sc_body(pts_hbm, w_hbm, b_hbm, out_hbm, slab, wv, bv, accv, sem):
    f32 = jnp.float32
    i32 = jnp.int32
    wid = lax.axis_index("s") * 2 + lax.axis_index("c")
    base = wid * PPW
    cp = pltpu.async_copy(pts_hbm.at[pl.ds(base * 4, PPW * 4)], slab, sem)
    pltpu.sync_copy(w_hbm, wv)
    pltpu.sync_copy(b_hbm, bv)

    zero16 = jnp.zeros((16,), f32)

    def zbody(i, c):
        accv[pl.ds(i * 16, 16)] = zero16
        return c
    lax.fori_loop(0, ACCN // 16, zbody, 0, unroll=8)
    cp.wait()

    wvs = [wv[pl.ds(j * 16, 16)] for j in range(8)]
    bvs = [bv[pl.ds(j * 16, 16)] for j in range(8)]

    lane = lax.broadcasted_iota(i32, (16,), 0)
    gy = jnp.minimum(lane + 1, 15)
    gi = jnp.minimum(lane + 3, 15)

    def gbody(v, c):
        q = slab[pl.ds(v * 16, 16)]
        yq = q.at[gy].get(mode="promise_in_bounds")
        iq = q.at[gi].get(mode="promise_in_bounds")
        cx = ((q + HALF) / RES).astype(i32)
        cy = ((yq + HALF) / RES).astype(i32)
        mask = (cx >= 0) & (cx < GRID) & (cy >= 0) & (cy < GRID)
        rowb = jnp.where(mask, (cx + 1) * PG + (cy + 1), PC) * 128
        for l in (0, 4, 8, 12):
            off = rowb[l]
            itv = jnp.broadcast_to(iq[l], (16,))
            for j in range(8):
                val = jnp.maximum(itv * wvs[j] + bvs[j], 0.0)
                plsc.addupdate(accv.at[pl.ds(off + j * 16, 16)], val)
        return c
    lax.fori_loop(0, PPW // 4, gbody, 0, unroll=2)

    pltpu.sync_copy(accv, out_hbm.at[wid])


def _sc_scatter(pts_flat, encw, encb):
    mesh = plsc.VectorSubcoreMesh(core_axis_name="c", subcore_axis_name="s")
    f32 = jnp.float32
    return pl.kernel(
        _sc_body,
        out_type=jax.ShapeDtypeStruct((NWORK, ACCN), f32),
        mesh=mesh,
        scratch_types=[
            pltpu.VMEM((PPW * 4,), f32),
            pltpu.VMEM((128,), f32),
            pltpu.VMEM((128,), f32),
            pltpu.VMEM((ACCN,), f32),
            pltpu.SemaphoreType.DMA,
        ],
    )(pts_flat, encw, encb)


def _shift_rows(g, s):
    n = g.shape[0]
    if s == 0:
        return g
    if s > 0:
        return jnp.concatenate([g[s:], g[:s]], axis=0)
    return jnp.concatenate([g[n + s:], g[:n + s]], axis=0)


def _tc_body(gp_ref, w1_ref, b1_ref, w2_ref, b2_ref,
             wh1_ref, bh1_ref,
             w2c_ref, b2c_ref, w2r_ref, b2r_ref, w2s_ref, b2s_ref, w2co_ref, b2co_ref,
             w3c_ref, b3c_ref, w3r_ref, b3r_ref, w3s_ref, b3s_ref, w3co_ref, b3co_ref,
             logits_ref, reg_ref, cyc_ref):
    f32 = jnp.float32
    i32 = jnp.int32

    g = gp_ref[0]
    for w in range(1, NWORK):
        g = g + gp_ref[w]

    riot = lax.broadcasted_iota(i32, (ROWS, 1), 0)
    xp = riot // PG
    yp = riot % PG
    interior = ((xp >= 1) & (xp <= GRID) & (yp >= 1) & (yp <= GRID)
                & (riot < PC)).astype(f32)

    bf16 = jnp.bfloat16
    hp = lax.Precision.HIGHEST
    gb = g.astype(bf16)
    h1 = jnp.zeros((ROWS, 64), f32)
    for k, s in enumerate(SHIFTS):
        h1 += lax.dot_general(
            _shift_rows(gb, s), w1_ref[k], (((1,), (0,)), ((), ())),
            preferred_element_type=f32)
    h1 = jnp.maximum(h1 + b1_ref[...], 0.0) * interior

    h1b = h1.astype(bf16)
    h2 = jnp.zeros((ROWS, 32), f32)
    for k, s in enumerate(SHIFTS):
        h2 += lax.dot_general(
            _shift_rows(h1b, s), w2_ref[k], (((1,), (0,)), ((), ())),
            preferred_element_type=f32)
    h2 = jnp.maximum(h2 + b2_ref[...], 0.0)

    br = lax.broadcasted_iota(i32, (36, ROWS), 0)
    rr = lax.broadcasted_iota(i32, (36, ROWS), 1)
    rxp = rr // PG
    ryp = rr % PG
    rvalid = (rxp >= 1) & (rxp <= GRID) & (ryp >= 1) & (ryp <= GRID) & (rr < PC)
    rblk = ((rxp - 1) // 4) * 6 + (ryp - 1) // 4
    pool = jnp.where((br == rblk) & rvalid, 1.0 / 16.0, 0.0).astype(f32)
    emb = lax.dot_general(pool, h2, (((1,), (0,)), ((), ())),
                          preferred_element_type=f32, precision=hp)

    embb = emb.astype(bf16)
    h = jnp.zeros((1, 512), f32)
    for ch in range(32):
        h += lax.dot_general(
            embb[:, ch:ch + 1], wh1_ref[ch], (((0,), (0,)), ((), ())),
            preferred_element_type=f32)
    h = jnp.maximum(h + bh1_ref[...], 0.0)

    def mm(a, w, b):
        return lax.dot_general(a.astype(bf16), w, (((1,), (0,)), ((), ())),
                               preferred_element_type=f32) + b

    hc = jnp.maximum(mm(h[:, 0:128], w2c_ref[...], b2c_ref[...]), 0.0)
    hr = jnp.maximum(mm(h[:, 128:256], w2r_ref[...], b2r_ref[...]), 0.0)
    hs = jnp.maximum(mm(h[:, 256:384], w2s_ref[...], b2s_ref[...]), 0.0)
    ho = jnp.maximum(mm(h[:, 384:512], w2co_ref[...], b2co_ref[...]), 0.0)

    logits_ref[...] = mm(hc, w3c_ref[...], b3c_ref[...])
    reg_ref[...] = mm(hr, w3r_ref[...], b3r_ref[...])
    sin_o = jnp.tanh(mm(hs, w3s_ref[...], b3s_ref[...]))
    cos_o = jnp.tanh(mm(ho, w3co_ref[...], b3co_ref[...]))
    cyc_ref[...] = jnp.concatenate([sin_o, cos_o], axis=1)


def kernel(ventana, params):
    f32 = jnp.float32
    pts_flat = ventana.reshape(-1)

    encw = params["enc"][0].reshape(128)
    encb = params["enc"][1].reshape(128)

    partials = _sc_scatter(pts_flat, encw, encb)
    gp = partials.reshape(NWORK, ROWS, 128)

    bf16 = jnp.bfloat16
    w1 = jnp.transpose(params["conv1"][0], (2, 3, 1, 0)).reshape(9, 128, 64).astype(bf16)
    b1 = params["conv1"][1].reshape(1, 64)
    w2 = jnp.transpose(params["conv2"][0], (2, 3, 1, 0)).reshape(9, 64, 32).astype(bf16)
    b2 = params["conv2"][1].reshape(1, 32)

    wh1 = jnp.concatenate([params[k][0][0] for k in ("clf", "reg", "sin", "cos")],
                          axis=1).reshape(32, 36, 512).astype(bf16)
    bh1 = jnp.concatenate([params[k][0][1] for k in ("clf", "reg", "sin", "cos")]
                          ).reshape(1, 512)

    def l2(k):
        return params[k][1][0].astype(bf16), params[k][1][1].reshape(1, -1)
    def l3(k):
        return params[k][2][0].astype(bf16), params[k][2][1].reshape(1, -1)

    w2c, b2c = l2("clf"); w2r, b2r = l2("reg")
    w2s, b2s = l2("sin"); w2co, b2co = l2("cos")
    w3c, b3c = l3("clf"); w3r, b3r = l3("reg")
    w3s, b3s = l3("sin"); w3co, b3co = l3("cos")

    out = pl.pallas_call(
        _tc_body,
        out_shape=(
            jax.ShapeDtypeStruct((1, 8), f32),
            jax.ShapeDtypeStruct((1, 6), f32),
            jax.ShapeDtypeStruct((1, 2), f32),
        ),
    )(gp, w1, b1, w2, b2, wh1, bh1,
      w2c, b2c, w2r, b2r, w2s, b2s, w2co, b2co,
      w3c, b3c, w3r, b3r, w3s, b3s, w3co, b3co)
    return out


# speedup vs baseline: 1.9445x; 1.9445x over previous
import jax
import jax.numpy as jnp
from jax import lax
from jax.experimental import pallas as pl
from jax.experimental.pallas import tpu as pltpu
from jax.experimental.pallas import tpu_sc as plsc

GRID = 24
RES = 0.25
HALF = GRID * RES / 2.0
PG = GRID + 2
PC = PG * PG
ROWS = 768
ACCN = ROWS * 128
NPTS = 16 * 2048
NWORK = 32
PPW = NPTS // NWORK
STRIPE = ROWS // 16

SHIFTS = [(kh - 1) * PG + (kw - 1) for kh in range(3) for kw in range(3)]


SMCH = 512


def _sc_body(xs_hbm, ys_hbm, is_hbm, w_hbm, b_hbm, out_hbm,
             xv, yv, iv, wv, bv, idxb, accv, shared, sem):
    f32 = jnp.float32
    i32 = jnp.int32
    sid = lax.axis_index("s")
    cid = lax.axis_index("c")
    wid = sid * 2 + cid
    base = wid * PPW
    cps = [pltpu.async_copy(xs_hbm.at[pl.ds(base, PPW)], xv, sem),
           pltpu.async_copy(ys_hbm.at[pl.ds(base, PPW)], yv, sem),
           pltpu.async_copy(is_hbm.at[pl.ds(base, PPW)], iv, sem)]
    pltpu.sync_copy(w_hbm, wv)
    pltpu.sync_copy(b_hbm, bv)

    lane = lax.broadcasted_iota(i32, (16,), 0)
    zero16 = jnp.zeros((16,), f32)

    def zbody(i, c):
        accv[i, pl.ds(0, 16)] = zero16
        accv[i, pl.ds(16, 16)] = zero16
        accv[i, pl.ds(32, 16)] = zero16
        accv[i, pl.ds(48, 16)] = zero16
        accv[i, pl.ds(64, 16)] = zero16
        accv[i, pl.ds(80, 16)] = zero16
        accv[i, pl.ds(96, 16)] = zero16
        accv[i, pl.ds(112, 16)] = zero16
        return c
    lax.fori_loop(0, ROWS, zbody, 0, unroll=4)

    for c in range(6):
        for k in range(8):
            idxb[c, pl.ds(k * 16, 16)] = lane + (c * 128 + k * 16)

    pltpu.sync_copy(accv.at[pl.ds(sid * STRIPE, STRIPE)],
                    shared.at[pl.ds(sid * STRIPE, STRIPE)])
    plsc.subcore_barrier()

    for cp in cps:
        cp.wait()

    wvs = [wv[pl.ds(j * 16, 16)] for j in range(8)]
    bvs = [bv[pl.ds(j * 16, 16)] for j in range(8)]

    def gbody(v, c):
        sl = pl.ds(v * 16, 16)
        x = xv[sl]
        y = yv[sl]
        it = iv[sl]
        cx = ((x + HALF) / RES).astype(i32)
        cy = ((y + HALF) / RES).astype(i32)
        mask = (cx >= 0) & (cx < GRID) & (cy >= 0) & (cy < GRID)
        rowb = jnp.where(mask, (cx + 1) * PG + (cy + 1), PC)
        for l in range(16):
            off = rowb[l]
            itv = jnp.broadcast_to(it[l], (16,))
            for j in range(8):
                val = jnp.maximum(itv * wvs[j] + bvs[j], 0.0)
                plsc.addupdate(accv.at[off, pl.ds(j * 16, 16)], val)
        return c
    lax.fori_loop(0, PPW // 16, gbody, 0)

    for c in range(6):
        pltpu.sync_copy(accv.at[pl.ds(c * 128, 128)],
                        shared.at[idxb.at[c]], add=True)
    plsc.subcore_barrier()

    pltpu.sync_copy(shared.at[pl.ds(sid * STRIPE, STRIPE)],
                    out_hbm.at[cid, pl.ds(sid * STRIPE, STRIPE)])


def _sc_scatter(xs, ys, iss, encw, encb):
    mesh = plsc.VectorSubcoreMesh(core_axis_name="c", subcore_axis_name="s")
    f32 = jnp.float32
    i32 = jnp.int32
    return pl.kernel(
        _sc_body,
        out_type=jax.ShapeDtypeStruct((2, ROWS, 128), f32),
        mesh=mesh,
        scratch_types=[
            pltpu.VMEM((PPW,), f32),
            pltpu.VMEM((PPW,), f32),
            pltpu.VMEM((PPW,), f32),
            pltpu.VMEM((128,), f32),
            pltpu.VMEM((128,), f32),
            pltpu.VMEM((6, 128), i32),
            pltpu.VMEM((ROWS, 128), f32),
            pltpu.VMEM_SHARED((ROWS, 128), f32),
            pltpu.SemaphoreType.DMA,
        ],
    )(xs, ys, iss, encw, encb)


def _shift_rows(g, s):
    n = g.shape[0]
    if s == 0:
        return g
    if s > 0:
        return jnp.concatenate([g[s:], g[:s]], axis=0)
    return jnp.concatenate([g[n + s:], g[:n + s]], axis=0)


def _tc_body(gp_ref, w1_ref, b1_ref, w2_ref, b2_ref,
             wh1_ref, bh1_ref,
             w2c_ref, b2c_ref, w2r_ref, b2r_ref, w2s_ref, b2s_ref, w2co_ref, b2co_ref,
             w3c_ref, b3c_ref, w3r_ref, b3r_ref, w3s_ref, b3s_ref, w3co_ref, b3co_ref,
             logits_ref, reg_ref, cyc_ref):
    f32 = jnp.float32
    i32 = jnp.int32

    g = gp_ref[0] + gp_ref[1]

    riot = lax.broadcasted_iota(i32, (ROWS, 1), 0)
    xp = riot // PG
    yp = riot % PG
    interior = ((xp >= 1) & (xp <= GRID) & (yp >= 1) & (yp <= GRID)
                & (riot < PC)).astype(f32)

    bf16 = jnp.bfloat16
    hp = lax.Precision.HIGHEST
    gb = g.astype(bf16)
    h1 = jnp.zeros((ROWS, 64), f32)
    for k, s in enumerate(SHIFTS):
        h1 += lax.dot_general(
            _shift_rows(gb, s), w1_ref[k], (((1,), (0,)), ((), ())),
            preferred_element_type=f32)
    h1 = jnp.maximum(h1 + b1_ref[...], 0.0) * interior

    h1b = h1.astype(bf16)
    h2 = jnp.zeros((ROWS, 32), f32)
    for k, s in enumerate(SHIFTS):
        h2 += lax.dot_general(
            _shift_rows(h1b, s), w2_ref[k], (((1,), (0,)), ((), ())),
            preferred_element_type=f32)
    h2 = jnp.maximum(h2 + b2_ref[...], 0.0)

    br = lax.broadcasted_iota(i32, (36, ROWS), 0)
    rr = lax.broadcasted_iota(i32, (36, ROWS), 1)
    rxp = rr // PG
    ryp = rr % PG
    rvalid = (rxp >= 1) & (rxp <= GRID) & (ryp >= 1) & (ryp <= GRID) & (rr < PC)
    rblk = ((rxp - 1) // 4) * 6 + (ryp - 1) // 4
    pool = jnp.where((br == rblk) & rvalid, 1.0 / 16.0, 0.0).astype(f32)
    emb = lax.dot_general(pool, h2, (((1,), (0,)), ((), ())),
                          preferred_element_type=f32, precision=hp)

    embb = emb.astype(bf16)
    h = jnp.zeros((1, 512), f32)
    for ch in range(32):
        h += lax.dot_general(
            embb[:, ch:ch + 1], wh1_ref[ch], (((0,), (0,)), ((), ())),
            preferred_element_type=f32)
    h = jnp.maximum(h + bh1_ref[...], 0.0)

    def mm(a, w, b):
        return lax.dot_general(a.astype(bf16), w, (((1,), (0,)), ((), ())),
                               preferred_element_type=f32) + b

    hc = jnp.maximum(mm(h[:, 0:128], w2c_ref[...], b2c_ref[...]), 0.0)
    hr = jnp.maximum(mm(h[:, 128:256], w2r_ref[...], b2r_ref[...]), 0.0)
    hs = jnp.maximum(mm(h[:, 256:384], w2s_ref[...], b2s_ref[...]), 0.0)
    ho = jnp.maximum(mm(h[:, 384:512], w2co_ref[...], b2co_ref[...]), 0.0)

    logits_ref[...] = mm(hc, w3c_ref[...], b3c_ref[...])
    reg_ref[...] = mm(hr, w3r_ref[...], b3r_ref[...])
    sin_o = jnp.tanh(mm(hs, w3s_ref[...], b3s_ref[...]))
    cos_o = jnp.tanh(mm(ho, w3co_ref[...], b3co_ref[...]))
    cyc_ref[...] = jnp.concatenate([sin_o, cos_o], axis=1)


def kernel(ventana, params):
    f32 = jnp.float32
    vt = ventana.reshape(-1, 4).T

    encw = params["enc"][0].reshape(128)
    encb = params["enc"][1].reshape(128)

    gp = _sc_scatter(vt[0], vt[1], vt[3], encw, encb)

    bf16 = jnp.bfloat16
    w1 = jnp.transpose(params["conv1"][0], (2, 3, 1, 0)).reshape(9, 128, 64).astype(bf16)
    b1 = params["conv1"][1].reshape(1, 64)
    w2 = jnp.transpose(params["conv2"][0], (2, 3, 1, 0)).reshape(9, 64, 32).astype(bf16)
    b2 = params["conv2"][1].reshape(1, 32)

    wh1 = jnp.concatenate([params[k][0][0] for k in ("clf", "reg", "sin", "cos")],
                          axis=1).reshape(32, 36, 512).astype(bf16)
    bh1 = jnp.concatenate([params[k][0][1] for k in ("clf", "reg", "sin", "cos")]
                          ).reshape(1, 512)

    def l2(k):
        return params[k][1][0].astype(bf16), params[k][1][1].reshape(1, -1)
    def l3(k):
        return params[k][2][0].astype(bf16), params[k][2][1].reshape(1, -1)

    w2c, b2c = l2("clf"); w2r, b2r = l2("reg")
    w2s, b2s = l2("sin"); w2co, b2co = l2("cos")
    w3c, b3c = l3("clf"); w3r, b3r = l3("reg")
    w3s, b3s = l3("sin"); w3co, b3co = l3("cos")

    out = pl.pallas_call(
        _tc_body,
        out_shape=(
            jax.ShapeDtypeStruct((1, 8), f32),
            jax.ShapeDtypeStruct((1, 6), f32),
            jax.ShapeDtypeStruct((1, 2), f32),
        ),
    )(gp, w1, b1, w2, b2, wh1, bh1,
      w2c, b2c, w2r, b2r, w2s, b2s, w2co, b2co,
      w3c, b3c, w3r, b3r, w3s, b3s, w3co, b3co)
    return out


# speedup vs baseline: 1.9497x; 1.0027x over previous
import jax
import jax.numpy as jnp
from jax import lax
from jax.experimental import pallas as pl
from jax.experimental.pallas import tpu as pltpu
from jax.experimental.pallas import tpu_sc as plsc

GRID = 24
RES = 0.25
HALF = GRID * RES / 2.0
PG = GRID + 2
PC = PG * PG
ROWS = 768
ACCN = ROWS * 128
NPTS = 16 * 2048
NWORK = 32
PPW = NPTS // NWORK
STRIPE = ROWS // 16

SHIFTS = [(kh - 1) * PG + (kw - 1) for kh in range(3) for kw in range(3)]


SMCH = 512


def _sc_body(xs_hbm, ys_hbm, is_hbm, w_hbm, b_hbm, out_hbm,
             xv, yv, iv, wv, bv, idxb, accv, shared, sem):
    f32 = jnp.float32
    i32 = jnp.int32
    sid = lax.axis_index("s")
    cid = lax.axis_index("c")
    wid = sid * 2 + cid
    base = wid * PPW
    cps = [pltpu.async_copy(xs_hbm.at[pl.ds(base, PPW)], xv, sem),
           pltpu.async_copy(ys_hbm.at[pl.ds(base, PPW)], yv, sem),
           pltpu.async_copy(is_hbm.at[pl.ds(base, PPW)], iv, sem)]
    pltpu.sync_copy(w_hbm, wv)
    pltpu.sync_copy(b_hbm, bv)

    lane = lax.broadcasted_iota(i32, (16,), 0)
    zero16 = jnp.zeros((16,), f32)

    @plsc.parallel_loop(0, ROWS, step=1)
    def _(i):
        accv[i, pl.ds(0, 16)] = zero16
        accv[i, pl.ds(16, 16)] = zero16
        accv[i, pl.ds(32, 16)] = zero16
        accv[i, pl.ds(48, 16)] = zero16
        accv[i, pl.ds(64, 16)] = zero16
        accv[i, pl.ds(80, 16)] = zero16
        accv[i, pl.ds(96, 16)] = zero16
        accv[i, pl.ds(112, 16)] = zero16

    for c in range(6):
        for k in range(8):
            idxb[c, pl.ds(k * 16, 16)] = lane + (c * 128 + k * 16)

    pltpu.sync_copy(accv.at[pl.ds(sid * STRIPE, STRIPE)],
                    shared.at[pl.ds(sid * STRIPE, STRIPE)])
    plsc.subcore_barrier()

    for cp in cps:
        cp.wait()

    wvs = [wv[pl.ds(j * 16, 16)] for j in range(8)]
    bvs = [bv[pl.ds(j * 16, 16)] for j in range(8)]

    @plsc.parallel_loop(0, PPW // 16, step=1)
    def _(v):
        sl = pl.ds(v * 16, 16)
        x = xv[sl]
        y = yv[sl]
        it = iv[sl]
        cx = ((x + HALF) / RES).astype(i32)
        cy = ((y + HALF) / RES).astype(i32)
        mask = (cx >= 0) & (cx < GRID) & (cy >= 0) & (cy < GRID)
        rowb = jnp.where(mask, (cx + 1) * PG + (cy + 1), PC)
        for l in range(16):
            off = rowb[l]
            itv = jnp.broadcast_to(it[l], (16,))
            for j in range(8):
                val = jnp.maximum(itv * wvs[j] + bvs[j], 0.0)
                plsc.addupdate(accv.at[off, pl.ds(j * 16, 16)], val)

    for c in range(6):
        pltpu.sync_copy(accv.at[pl.ds(c * 128, 128)],
                        shared.at[idxb.at[c]], add=True)
    plsc.subcore_barrier()

    pltpu.sync_copy(shared.at[pl.ds(sid * STRIPE, STRIPE)],
                    out_hbm.at[cid, pl.ds(sid * STRIPE, STRIPE)])


def _sc_scatter(xs, ys, iss, encw, encb):
    mesh = plsc.VectorSubcoreMesh(core_axis_name="c", subcore_axis_name="s")
    f32 = jnp.float32
    i32 = jnp.int32
    return pl.kernel(
        _sc_body,
        out_type=jax.ShapeDtypeStruct((2, ROWS, 128), f32),
        mesh=mesh,
        scratch_types=[
            pltpu.VMEM((PPW,), f32),
            pltpu.VMEM((PPW,), f32),
            pltpu.VMEM((PPW,), f32),
            pltpu.VMEM((128,), f32),
            pltpu.VMEM((128,), f32),
            pltpu.VMEM((6, 128), i32),
            pltpu.VMEM((ROWS, 128), f32),
            pltpu.VMEM_SHARED((ROWS, 128), f32),
            pltpu.SemaphoreType.DMA,
        ],
    )(xs, ys, iss, encw, encb)


def _shift_rows(g, s):
    n = g.shape[0]
    if s == 0:
        return g
    if s > 0:
        return jnp.concatenate([g[s:], g[:s]], axis=0)
    return jnp.concatenate([g[n + s:], g[:n + s]], axis=0)


def _tc_body(gp_ref, w1_ref, b1_ref, w2_ref, b2_ref,
             wh1_ref, bh1_ref,
             w2c_ref, b2c_ref, w2r_ref, b2r_ref, w2s_ref, b2s_ref, w2co_ref, b2co_ref,
             w3c_ref, b3c_ref, w3r_ref, b3r_ref, w3s_ref, b3s_ref, w3co_ref, b3co_ref,
             logits_ref, reg_ref, cyc_ref):
    f32 = jnp.float32
    i32 = jnp.int32

    g = gp_ref[0] + gp_ref[1]

    riot = lax.broadcasted_iota(i32, (ROWS, 1), 0)
    xp = riot // PG
    yp = riot % PG
    interior = ((xp >= 1) & (xp <= GRID) & (yp >= 1) & (yp <= GRID)
                & (riot < PC)).astype(f32)

    bf16 = jnp.bfloat16
    hp = lax.Precision.HIGHEST
    gb = g.astype(bf16)
    h1 = jnp.zeros((ROWS, 64), f32)
    for k, s in enumerate(SHIFTS):
        h1 += lax.dot_general(
            _shift_rows(gb, s), w1_ref[k], (((1,), (0,)), ((), ())),
            preferred_element_type=f32)
    h1 = jnp.maximum(h1 + b1_ref[...], 0.0) * interior

    h1b = h1.astype(bf16)
    h2 = jnp.zeros((ROWS, 32), f32)
    for k, s in enumerate(SHIFTS):
        h2 += lax.dot_general(
            _shift_rows(h1b, s), w2_ref[k], (((1,), (0,)), ((), ())),
            preferred_element_type=f32)
    h2 = jnp.maximum(h2 + b2_ref[...], 0.0)

    br = lax.broadcasted_iota(i32, (36, ROWS), 0)
    rr = lax.broadcasted_iota(i32, (36, ROWS), 1)
    rxp = rr // PG
    ryp = rr % PG
    rvalid = (rxp >= 1) & (rxp <= GRID) & (ryp >= 1) & (ryp <= GRID) & (rr < PC)
    rblk = ((rxp - 1) // 4) * 6 + (ryp - 1) // 4
    pool = jnp.where((br == rblk) & rvalid, 1.0 / 16.0, 0.0).astype(f32)
    emb = lax.dot_general(pool, h2, (((1,), (0,)), ((), ())),
                          preferred_element_type=f32, precision=hp)

    embb = emb.astype(bf16)
    h = jnp.zeros((1, 512), f32)
    for ch in range(32):
        h += lax.dot_general(
            embb[:, ch:ch + 1], wh1_ref[ch], (((0,), (0,)), ((), ())),
            preferred_element_type=f32)
    h = jnp.maximum(h + bh1_ref[...], 0.0)

    def mm(a, w, b):
        return lax.dot_general(a.astype(bf16), w, (((1,), (0,)), ((), ())),
                               preferred_element_type=f32) + b

    hc = jnp.maximum(mm(h[:, 0:128], w2c_ref[...], b2c_ref[...]), 0.0)
    hr = jnp.maximum(mm(h[:, 128:256], w2r_ref[...], b2r_ref[...]), 0.0)
    hs = jnp.maximum(mm(h[:, 256:384], w2s_ref[...], b2s_ref[...]), 0.0)
    ho = jnp.maximum(mm(h[:, 384:512], w2co_ref[...], b2co_ref[...]), 0.0)

    logits_ref[...] = mm(hc, w3c_ref[...], b3c_ref[...])
    reg_ref[...] = mm(hr, w3r_ref[...], b3r_ref[...])
    sin_o = jnp.tanh(mm(hs, w3s_ref[...], b3s_ref[...]))
    cos_o = jnp.tanh(mm(ho, w3co_ref[...], b3co_ref[...]))
    cyc_ref[...] = jnp.concatenate([sin_o, cos_o], axis=1)


def kernel(ventana, params):
    f32 = jnp.float32
    vt = ventana.reshape(-1, 4).T

    encw = params["enc"][0].reshape(128)
    encb = params["enc"][1].reshape(128)

    gp = _sc_scatter(vt[0], vt[1], vt[3], encw, encb)

    bf16 = jnp.bfloat16
    w1 = jnp.transpose(params["conv1"][0], (2, 3, 1, 0)).reshape(9, 128, 64).astype(bf16)
    b1 = params["conv1"][1].reshape(1, 64)
    w2 = jnp.transpose(params["conv2"][0], (2, 3, 1, 0)).reshape(9, 64, 32).astype(bf16)
    b2 = params["conv2"][1].reshape(1, 32)

    wh1 = jnp.concatenate([params[k][0][0] for k in ("clf", "reg", "sin", "cos")],
                          axis=1).reshape(32, 36, 512).astype(bf16)
    bh1 = jnp.concatenate([params[k][0][1] for k in ("clf", "reg", "sin", "cos")]
                          ).reshape(1, 512)

    def l2(k):
        return params[k][1][0].astype(bf16), params[k][1][1].reshape(1, -1)
    def l3(k):
        return params[k][2][0].astype(bf16), params[k][2][1].reshape(1, -1)

    w2c, b2c = l2("clf"); w2r, b2r = l2("reg")
    w2s, b2s = l2("sin"); w2co, b2co = l2("cos")
    w3c, b3c = l3("clf"); w3r, b3r = l3("reg")
    w3s, b3s = l3("sin"); w3co, b3co = l3("cos")

    out = pl.pallas_call(
        _tc_body,
        out_shape=(
            jax.ShapeDtypeStruct((1, 8), f32),
            jax.ShapeDtypeStruct((1, 6), f32),
            jax.ShapeDtypeStruct((1, 2), f32),
        ),
    )(gp, w1, b1, w2, b2, wh1, bh1,
      w2c, b2c, w2r, b2r, w2s, b2s, w2co, b2co,
      w3c, b3c, w3r, b3r, w3s, b3s, w3co, b3co)
    return out
